# Initial kernel scaffold; baseline (speedup 1.0000x reference)
import jax
import jax.numpy as jnp
from jax.experimental import pallas as pl


def kernel(x_raw, embed_last, wmat0, bias0, wmat1, bias1, wmat2, bias2):
    raise NotImplementedError("write your pallas kernel here")


# speedup vs baseline: 23.4615x; 23.4615x over previous
import jax
import jax.numpy as jnp
from jax.experimental import pallas as pl
from jax.experimental.pallas import tpu as pltpu


def _fused_head_kernel(x_ref, w1_ref, b1_ref, w2_ref, b2_ref, w3_ref, b3_ref,
                       o_ref, acc_ref, pad_ref):
    t = pl.program_id(1)
    kd = t // 9
    kh = (t // 3) % 3
    kw = t % 3

    @pl.when(t == 0)
    def _():
        acc_ref[...] = jnp.zeros_like(acc_ref)

    a = x_ref[0, kd % 2, kh % 2, kw % 2,
              pl.ds(kd // 2, 4), pl.ds(kh // 2, 4), pl.ds(kw // 2, 4), :, :]
    a = a.reshape(512, 768)
    acc_ref[...] += jnp.dot(a, w1_ref[0], preferred_element_type=jnp.float32)

    @pl.when(t == 26)
    def _():
        h1 = jnp.maximum(acc_ref[...] + b1_ref[...], 0.0).astype(jnp.bfloat16)
        pad_ref[...] = jnp.zeros_like(pad_ref)
        pad_ref[1:5, 1:5, 1:5, :, :] = h1.reshape(4, 4, 4, 8, 512)

        pv = pad_ref[...].reshape(3, 2, 3, 2, 3, 2, 8, 512)
        acc2 = jnp.zeros((64, 256), jnp.float32)
        for dz in range(3):
            for dy in range(3):
                for dx in range(3):
                    aa = pv[dz // 2:dz // 2 + 2, dz % 2,
                            dy // 2:dy // 2 + 2, dy % 2,
                            dx // 2:dx // 2 + 2, dx % 2, :, :]
                    acc2 += jnp.dot(aa.reshape(64, 512),
                                    w2_ref[dz * 9 + dy * 3 + dx],
                                    preferred_element_type=jnp.float32)
        h2 = jnp.maximum(acc2 + b2_ref[...], 0.0).astype(jnp.bfloat16)
        h2 = h2.reshape(2, 2, 2, 8, 256)

        acc3 = jnp.zeros((8, 128), jnp.float32)
        for dz in range(1, 3):
            for dy in range(1, 3):
                for dx in range(1, 3):
                    acc3 += jnp.dot(h2[dz - 1, dy - 1, dx - 1],
                                    w3_ref[dz * 9 + dy * 3 + dx],
                                    preferred_element_type=jnp.float32)
        emb = acc3 + b3_ref[...]

        ss = jnp.sum(emb * emb, axis=1, keepdims=True)
        o_ref[...] = emb * jax.lax.rsqrt(jnp.maximum(ss, 1e-24))


def kernel(x_raw, embed_last, wmat0, bias0, wmat1, bias1, wmat2, bias2):
    del x_raw

    x = embed_last.astype(jnp.bfloat16)
    x = jnp.transpose(x, (0, 2, 3, 4, 1))
    x = jnp.pad(x, ((0, 0), (1, 1), (1, 1), (1, 1), (0, 0)))
    x = x.reshape(2, 8, 5, 2, 5, 2, 5, 2, 768)
    x = jnp.transpose(x, (0, 3, 5, 7, 2, 4, 6, 1, 8))

    w1 = wmat0.reshape(27, 768, 512)
    w2 = wmat1.reshape(27, 512, 256)
    w3 = wmat2.reshape(27, 256, 128)

    return pl.pallas_call(
        _fused_head_kernel,
        out_shape=jax.ShapeDtypeStruct((16, 128), jnp.float32),
        grid=(2, 27),
        in_specs=[
            pl.BlockSpec((1, 2, 2, 2, 5, 5, 5, 8, 768),
                         lambda b, t: (b, 0, 0, 0, 0, 0, 0, 0, 0)),
            pl.BlockSpec((1, 768, 512), lambda b, t: (t, 0, 0)),
            pl.BlockSpec((1, 512), lambda b, t: (0, 0)),
            pl.BlockSpec((27, 512, 256), lambda b, t: (0, 0, 0)),
            pl.BlockSpec((1, 256), lambda b, t: (0, 0)),
            pl.BlockSpec((27, 256, 128), lambda b, t: (0, 0, 0)),
            pl.BlockSpec((1, 128), lambda b, t: (0, 0)),
        ],
        out_specs=pl.BlockSpec((8, 128), lambda b, t: (b, 0)),
        scratch_shapes=[
            pltpu.VMEM((512, 512), jnp.float32),
            pltpu.VMEM((6, 6, 6, 8, 512), jnp.bfloat16),
        ],
        compiler_params=pltpu.CompilerParams(
            dimension_semantics=("parallel", "arbitrary"),
            vmem_limit_bytes=56 * 1024 * 1024),
        name="fused_get_embed_head",
    )(x, w1, bias0, w2, bias1, w3, bias2)


# speedup vs baseline: 37.1377x; 1.5829x over previous
import jax
import jax.numpy as jnp
from jax.experimental import pallas as pl
from jax.experimental.pallas import tpu as pltpu


def _fused_head_kernel(x_hbm, w1_ref, b1_ref, w2_ref, b2_ref, w3_ref, b3_ref,
                       o_ref, acc_ref, xpad_ref, pad2_ref, sem):
    b = pl.program_id(0)
    t = pl.program_id(1)
    kd = t // 3
    kh = t % 3

    @pl.when(t == 0)
    def _():
        acc_ref[...] = jnp.zeros_like(acc_ref)
        xpad_ref[0, :, :, 0, :, :, :, :] = jnp.zeros_like(
            xpad_ref[0, :, :, 0, :, :, :, :])
        xpad_ref[1, :, :, 4, :, :, :, :] = jnp.zeros_like(
            xpad_ref[1, :, :, 4, :, :, :, :])
        xpad_ref[:, 0, :, :, 0, :, :, :] = jnp.zeros_like(
            xpad_ref[:, 0, :, :, 0, :, :, :])
        xpad_ref[:, 1, :, :, 4, :, :, :] = jnp.zeros_like(
            xpad_ref[:, 1, :, :, 4, :, :, :])
        xpad_ref[:, :, 0, :, :, 0, :, :] = jnp.zeros_like(
            xpad_ref[:, :, 0, :, :, 0, :, :])
        xpad_ref[:, :, 1, :, :, 4, :, :] = jnp.zeros_like(
            xpad_ref[:, :, 1, :, :, 4, :, :])
        for jd in range(2):
            for jh in range(2):
                for jw in range(2):
                    pltpu.make_async_copy(
                        x_hbm.at[b, jd, jh, jw],
                        xpad_ref.at[1 - jd, 1 - jh, 1 - jw,
                                    pl.ds(jd, 4), pl.ds(jh, 4), pl.ds(jw, 4),
                                    :, :],
                        sem).start()
        for _ in range(8):
            pltpu.make_async_copy(
                x_hbm.at[0, 0, 0, 0],
                xpad_ref.at[0, 0, 0, pl.ds(0, 4), pl.ds(0, 4), pl.ds(0, 4),
                            :, :],
                sem).wait()

    for kw in range(3):
        a = xpad_ref[kd % 2, kh % 2, kw % 2,
                     pl.ds(kd // 2, 4), pl.ds(kh // 2, 4), pl.ds(kw // 2, 4),
                     :, :]
        a = a.reshape(512, 768)
        acc_ref[...] += jnp.dot(a, w1_ref[kw],
                                preferred_element_type=jnp.float32)

    @pl.when(t == 8)
    def _():
        h1 = jnp.maximum(acc_ref[...] + b1_ref[...], 0.0).astype(jnp.bfloat16)
        pad2_ref[...] = jnp.zeros_like(pad2_ref)
        pad2_ref[1:5, 1:5, 1:5, :, :] = h1.reshape(4, 4, 4, 8, 512)

        pv = pad2_ref[...].reshape(3, 2, 3, 2, 3, 2, 8, 512)
        acc2 = jnp.zeros((64, 256), jnp.float32)
        for dz in range(3):
            for dy in range(3):
                for dx in range(3):
                    aa = pv[dz // 2:dz // 2 + 2, dz % 2,
                            dy // 2:dy // 2 + 2, dy % 2,
                            dx // 2:dx // 2 + 2, dx % 2, :, :]
                    acc2 += jnp.dot(aa.reshape(64, 512),
                                    w2_ref[dz * 9 + dy * 3 + dx],
                                    preferred_element_type=jnp.float32)
        h2 = jnp.maximum(acc2 + b2_ref[...], 0.0).astype(jnp.bfloat16)
        h2 = h2.reshape(2, 2, 2, 8, 256)

        acc3 = jnp.zeros((8, 128), jnp.float32)
        for dz in range(1, 3):
            for dy in range(1, 3):
                for dx in range(1, 3):
                    acc3 += jnp.dot(h2[dz - 1, dy - 1, dx - 1],
                                    w3_ref[dz * 9 + dy * 3 + dx],
                                    preferred_element_type=jnp.float32)
        emb = acc3 + b3_ref[...]

        ss = jnp.sum(emb * emb, axis=1, keepdims=True)
        o_ref[...] = emb * jax.lax.rsqrt(jnp.maximum(ss, 1e-24))


def kernel(x_raw, embed_last, wmat0, bias0, wmat1, bias1, wmat2, bias2):
    del x_raw

    x = embed_last.astype(jnp.bfloat16)
    x = x.reshape(2, 8, 768, 4, 2, 4, 2, 4, 2)
    x = x.transpose(0, 4, 6, 8, 3, 5, 7, 1, 2)

    w1 = wmat0.reshape(27, 768, 512)
    w2 = wmat1.reshape(27, 512, 256)
    w3 = wmat2.reshape(27, 256, 128)

    return pl.pallas_call(
        _fused_head_kernel,
        out_shape=jax.ShapeDtypeStruct((16, 128), jnp.float32),
        grid=(2, 9),
        in_specs=[
            pl.BlockSpec(memory_space=pl.ANY),
            pl.BlockSpec((3, 768, 512), lambda b, t: (t, 0, 0)),
            pl.BlockSpec((1, 512), lambda b, t: (0, 0)),
            pl.BlockSpec((27, 512, 256), lambda b, t: (0, 0, 0)),
            pl.BlockSpec((1, 256), lambda b, t: (0, 0)),
            pl.BlockSpec((27, 256, 128), lambda b, t: (0, 0, 0)),
            pl.BlockSpec((1, 128), lambda b, t: (0, 0)),
        ],
        out_specs=pl.BlockSpec((8, 128), lambda b, t: (b, 0)),
        scratch_shapes=[
            pltpu.VMEM((512, 512), jnp.float32),
            pltpu.VMEM((2, 2, 2, 5, 5, 5, 8, 768), jnp.bfloat16),
            pltpu.VMEM((6, 6, 6, 8, 512), jnp.bfloat16),
            pltpu.SemaphoreType.DMA,
        ],
        compiler_params=pltpu.CompilerParams(
            dimension_semantics=("parallel", "arbitrary"),
            vmem_limit_bytes=56 * 1024 * 1024),
        name="fused_get_embed_head",
    )(x, w1, bias0, w2, bias1, w3, bias2)


# speedup vs baseline: 37.9791x; 1.0227x over previous
import jax
import jax.numpy as jnp
from jax.experimental import pallas as pl
from jax.experimental.pallas import tpu as pltpu


def _fused_head_kernel(x_hbm, w1_ref, b1_ref, w2_ref, b2_ref, w3_ref, b3_ref,
                       o_ref, acc_ref, xin_ref, xpad_ref, pad2_ref, sems):
    b = pl.program_id(0)
    t = pl.program_id(1)
    kd = t // 3
    kh = t % 3

    @pl.when(t == 0)
    def _():
        acc_ref[...] = jnp.zeros_like(acc_ref)
        for bb in range(8):
            pltpu.make_async_copy(x_hbm.at[b * 8 + bb], xin_ref.at[bb],
                                  sems.at[bb]).start()
        xpad_ref[0, :, :, :, 0, :, :, :] = jnp.zeros_like(
            xpad_ref[0, :, :, :, 0, :, :, :])
        xpad_ref[1, :, :, :, 4, :, :, :] = jnp.zeros_like(
            xpad_ref[1, :, :, :, 4, :, :, :])
        xpad_ref[:, 0, :, :, :, 0, :, :] = jnp.zeros_like(
            xpad_ref[:, 0, :, :, :, 0, :, :])
        xpad_ref[:, 1, :, :, :, 4, :, :] = jnp.zeros_like(
            xpad_ref[:, 1, :, :, :, 4, :, :])
        xpad_ref[:, :, 0, :, :, :, 0, :] = jnp.zeros_like(
            xpad_ref[:, :, 0, :, :, :, 0, :])
        xpad_ref[:, :, 1, :, :, :, 4, :] = jnp.zeros_like(
            xpad_ref[:, :, 1, :, :, :, 4, :])
        for bb in range(8):
            pltpu.make_async_copy(x_hbm.at[0], xin_ref.at[bb],
                                  sems.at[bb]).wait()
            xt = jnp.swapaxes(xin_ref[bb], 0, 1).astype(jnp.bfloat16)
            xt6 = xt.reshape(4, 2, 4, 2, 4, 2, 768)
            for jd in range(2):
                for jh in range(2):
                    for jw in range(2):
                        xpad_ref[1 - jd, 1 - jh, 1 - jw, bb,
                                 pl.ds(jd, 4), pl.ds(jh, 4), pl.ds(jw, 4),
                                 :] = xt6[:, jd, :, jh, :, jw, :]

    for kw in range(3):
        a = xpad_ref[kd % 2, kh % 2, kw % 2, :,
                     pl.ds(kd // 2, 4), pl.ds(kh // 2, 4), pl.ds(kw // 2, 4),
                     :]
        a = a.reshape(512, 768)
        acc_ref[...] += jnp.dot(a, w1_ref[kw],
                                preferred_element_type=jnp.float32)

    @pl.when(t == 8)
    def _():
        h1 = jnp.maximum(acc_ref[...] + b1_ref[...], 0.0).astype(jnp.bfloat16)
        pad2_ref[...] = jnp.zeros_like(pad2_ref)
        pad2_ref[:, 1:5, 1:5, 1:5, :] = h1.reshape(8, 4, 4, 4, 512)

        pv = pad2_ref[...].reshape(8, 3, 2, 3, 2, 3, 2, 512)
        acc2 = jnp.zeros((64, 256), jnp.float32)
        for dz in range(3):
            for dy in range(3):
                for dx in range(3):
                    aa = pv[:, dz // 2:dz // 2 + 2, dz % 2,
                            dy // 2:dy // 2 + 2, dy % 2,
                            dx // 2:dx // 2 + 2, dx % 2, :]
                    acc2 += jnp.dot(aa.reshape(64, 512),
                                    w2_ref[dz * 9 + dy * 3 + dx],
                                    preferred_element_type=jnp.float32)
        h2 = jnp.maximum(acc2 + b2_ref[...], 0.0).astype(jnp.bfloat16)
        h2 = h2.reshape(8, 2, 2, 2, 256)

        acc3 = jnp.zeros((8, 128), jnp.float32)
        for dz in range(1, 3):
            for dy in range(1, 3):
                for dx in range(1, 3):
                    acc3 += jnp.dot(h2[:, dz - 1, dy - 1, dx - 1, :],
                                    w3_ref[dz * 9 + dy * 3 + dx],
                                    preferred_element_type=jnp.float32)
        emb = acc3 + b3_ref[...]

        ss = jnp.sum(emb * emb, axis=1, keepdims=True)
        o_ref[...] = emb * jax.lax.rsqrt(jnp.maximum(ss, 1e-24))


def kernel(x_raw, embed_last, wmat0, bias0, wmat1, bias1, wmat2, bias2):
    del x_raw

    x = embed_last.reshape(16, 768, 512)

    w1 = wmat0.reshape(27, 768, 512)
    w2 = wmat1.reshape(27, 512, 256)
    w3 = wmat2.reshape(27, 256, 128)

    return pl.pallas_call(
        _fused_head_kernel,
        out_shape=jax.ShapeDtypeStruct((16, 128), jnp.float32),
        grid=(2, 9),
        in_specs=[
            pl.BlockSpec(memory_space=pl.ANY),
            pl.BlockSpec((3, 768, 512), lambda b, t: (t, 0, 0)),
            pl.BlockSpec((1, 512), lambda b, t: (0, 0)),
            pl.BlockSpec((27, 512, 256), lambda b, t: (0, 0, 0)),
            pl.BlockSpec((1, 256), lambda b, t: (0, 0)),
            pl.BlockSpec((27, 256, 128), lambda b, t: (0, 0, 0)),
            pl.BlockSpec((1, 128), lambda b, t: (0, 0)),
        ],
        out_specs=pl.BlockSpec((8, 128), lambda b, t: (b, 0)),
        scratch_shapes=[
            pltpu.VMEM((512, 512), jnp.float32),
            pltpu.VMEM((8, 768, 512), jnp.float32),
            pltpu.VMEM((2, 2, 2, 8, 5, 5, 5, 768), jnp.bfloat16),
            pltpu.VMEM((8, 6, 6, 6, 512), jnp.bfloat16),
            pltpu.SemaphoreType.DMA((8,)),
        ],
        compiler_params=pltpu.CompilerParams(
            dimension_semantics=("parallel", "arbitrary"),
            vmem_limit_bytes=56 * 1024 * 1024),
        name="fused_get_embed_head",
    )(x, w1, bias0, w2, bias1, w3, bias2)
